# baseline (speedup 1.0000x reference)
import functools

import jax
import jax.numpy as jnp
from jax import lax
from jax.experimental import pallas as pl
from jax.experimental.pallas import tpu as pltpu
from jax.experimental.pallas import tpu_sc as plsc

_N = 10000
_M = 10000
_E = 320000
_C = 128

_NC = 2
_NS = 16
_NW = _NC * _NS

_MP = 10240
_SL = _MP // _NS
_EP = 327680
_EB = _EP // _NW
_CK = 1024
_JS = _CK // 128
_CB = _EB // _CK
_EA = _EP // _NS
_CA = _EA // _CK

_f32 = jnp.float32
_i32 = jnp.int32


def _iota16():
  return lax.broadcasted_iota(_i32, (16,), 0)


def _zero16():
  return jnp.zeros((16,), _f32)


def _tc_pre(x, hattr, w, att_a, att_b):
  blk = 400
  nb = _N // blk

  def body(x_ref, h_ref, w_ref, aa_ref, ab_ref, xl_ref, ax_ref, ae_ref,
           sh_ref, smem):
    i = pl.program_id(0)

    @pl.when(i == 0)
    def _():
      smem[0] = -1e30
      smem[1] = -1e30

    wv = w_ref[...]
    xb = jnp.dot(x_ref[...], wv, preferred_element_type=_f32)
    axb = jnp.sum(xb * aa_ref[...], axis=1)
    eb = jnp.dot(h_ref[...], wv, preferred_element_type=_f32)
    aeb = jnp.sum(eb * ab_ref[...], axis=1)
    xl_ref[...] = xb
    ax_ref[...] = axb[:, None]
    ae_ref[...] = aeb[:, None]
    smem[0] = jnp.maximum(smem[0], jnp.max(axb))
    smem[1] = jnp.maximum(smem[1], jnp.max(aeb))

    @pl.when(i == nb - 1)
    def _():
      sh_ref[...] = jnp.full((16,), jnp.maximum(smem[0] + smem[1], 0.0), _f32)

  return pl.pallas_call(
      body,
      grid=(nb,),
      in_specs=[
          pl.BlockSpec((blk, _C), lambda i: (i, 0)),
          pl.BlockSpec((blk, _C), lambda i: (i, 0)),
          pl.BlockSpec((_C, _C), lambda i: (0, 0)),
          pl.BlockSpec((1, _C), lambda i: (0, 0)),
          pl.BlockSpec((1, _C), lambda i: (0, 0)),
      ],
      out_specs=[
          pl.BlockSpec((blk, _C), lambda i: (i, 0)),
          pl.BlockSpec((blk, 1), lambda i: (i, 0)),
          pl.BlockSpec((blk, 1), lambda i: (i, 0)),
          pl.BlockSpec((16,), lambda i: (0,)),
      ],
      out_shape=[
          jax.ShapeDtypeStruct((_N, _C), _f32),
          jax.ShapeDtypeStruct((_N, 1), _f32),
          jax.ShapeDtypeStruct((_M, 1), _f32),
          jax.ShapeDtypeStruct((16,), _f32),
      ],
      scratch_shapes=[pltpu.SMEM((2,), _f32)],
  )(x, hattr, w, att_a, att_b)


def _tc_combine(parts, den2, binv2):
  blk = 80
  nb = _MP // blk

  def body(a_ref, b_ref, da_ref, db_ref, bi_ref, o_ref):
    q = 1.0 / (da_ref[...] + db_ref[...] + 1e-16)
    o_ref[...] = (a_ref[...] + b_ref[...]) * (bi_ref[...] * q * q)

  return pl.pallas_call(
      body,
      grid=(nb,),
      in_specs=[
          pl.BlockSpec((blk, _C), lambda i: (i, 0)),
          pl.BlockSpec((blk, _C), lambda i: (i + nb, 0)),
          pl.BlockSpec((blk, 1), lambda i: (i, 0)),
          pl.BlockSpec((blk, 1), lambda i: (i + nb, 0)),
          pl.BlockSpec((blk, 1), lambda i: (i, 0)),
      ],
      out_specs=pl.BlockSpec((blk, _C), lambda i: (i, 0)),
      out_shape=jax.ShapeDtypeStruct((_MP, _C), _f32),
  )(parts, parts, den2, den2, binv2)


def _tc_post(parts, dinv2, bvec, gvec, bevec, x):
  blk = 80
  nb = _MP // blk

  def body(a_ref, b_ref, di_ref, bv_ref, g_ref, be_ref, x_ref, o_ref):
    o = (a_ref[...] + b_ref[...]) * di_ref[...] + bv_ref[...]
    m = jnp.mean(o, axis=1, keepdims=True)
    d = o - m
    v = jnp.mean(d * d, axis=1, keepdims=True)
    ln = d / jnp.sqrt(v + 1e-5) * g_ref[...] + be_ref[...]
    o_ref[...] = jnp.tanh(ln + x_ref[...])

  return pl.pallas_call(
      body,
      grid=(_N // blk,),
      in_specs=[
          pl.BlockSpec((blk, _C), lambda i: (i, 0)),
          pl.BlockSpec((blk, _C), lambda i: (i + nb, 0)),
          pl.BlockSpec((blk, 1), lambda i: (i, 0)),
          pl.BlockSpec((1, _C), lambda i: (0, 0)),
          pl.BlockSpec((1, _C), lambda i: (0, 0)),
          pl.BlockSpec((1, _C), lambda i: (0, 0)),
          pl.BlockSpec((blk, _C), lambda i: (i, 0)),
      ],
      out_specs=pl.BlockSpec((blk, _C), lambda i: (i, 0)),
      out_shape=jax.ShapeDtypeStruct((_N, _C), _f32),
  )(parts, parts, dinv2, bvec, gvec, bevec, x)


def _sc_mesh():
  return plsc.VectorSubcoreMesh(core_axis_name="c", subcore_axis_name="s")


def _sc_degrees():
  @functools.partial(
      pl.kernel,
      out_type=jax.ShapeDtypeStruct((2 * _MP,), _f32),
      mesh=_sc_mesh(),
      compiler_params=pltpu.CompilerParams(needs_layout_passes=False),
      scratch_types=[
          pltpu.VMEM((_JS, 128), _i32),
          pltpu.VMEM((_CK,), _f32),
          pltpu.VMEM((_JS, 128), _f32),
          pltpu.VMEM((_SL,), _f32),
          pltpu.VMEM_SHARED((_MP,), _f32),
          pltpu.SemaphoreType.DMA,
          pltpu.SemaphoreType.DMA,
      ],
  )
  def k(rc2_h, st2_h, out_h, idxb, stb, updb, tbuf, acc, lsem, ssem):
    c = lax.axis_index("c")
    s = lax.axis_index("s")
    iota = _iota16()
    cf = jnp.full((16,), (1 - c).astype(_f32), _f32)

    for g in range(_SL // 16):
      tbuf[pl.ds(g * 16, 16)] = _zero16()
    pltpu.sync_copy(tbuf, acc.at[pl.ds(s * _SL, _SL)])
    plsc.subcore_barrier()

    def fire_loads(base_e):
      pltpu.async_copy(st2_h.at[pl.ds(base_e, _CK)], stb, lsem)
      for j in range(_JS):
        pltpu.async_copy(rc2_h.at[pl.ds(c * _EP + base_e + j * 128, 128)],
                         idxb.at[j], lsem)

    def drain_loads(base_e):
      pltpu.make_async_copy(st2_h.at[pl.ds(base_e, _CK)], stb, lsem).wait()
      for j in range(_JS):
        pltpu.make_async_copy(rc2_h.at[pl.ds(c * _EP + base_e + j * 128, 128)],
                              idxb.at[j], lsem).wait()

    fire_loads(s * _EA)

    def chunk(k_, _):
      base_e = s * _EA + k_ * _CK
      drain_loads(base_e)
      for j in range(_JS):
        for g in range(8):
          off = j * 128 + g * 16
          msk = (base_e + off + iota) < _E
          sv = jnp.abs(stb[pl.ds(off, 16)])
          val = cf + (1.0 - cf) * sv
          updb[j, pl.ds(g * 16, 16)] = jnp.where(msk, val, 0.0)
      sd = []
      for j in range(_JS):
        sd.append(pltpu.async_copy(updb.at[j], acc.at[idxb.at[j]], ssem,
                                   add=True))
      for d in sd:
        d.wait()
      nxt = s * _EA + jnp.minimum(k_ + 1, _CA - 1) * _CK
      fire_loads(nxt)
      return 0

    lax.fori_loop(0, _CA, chunk, 0)
    drain_loads(s * _EA + (_CA - 1) * _CK)
    plsc.subcore_barrier()

    pltpu.sync_copy(acc.at[pl.ds(s * _SL, _SL)], tbuf)
    for g in range(_SL // 16):
      v = tbuf[pl.ds(g * 16, 16)]
      tbuf[pl.ds(g * 16, 16)] = jnp.where(v > 0, 1.0 / v, 0.0)
    pltpu.sync_copy(tbuf, out_h.at[pl.ds(c * _MP + s * _SL, _SL)])

  return k


def _sc_pass1(do_deg):
  @functools.partial(
      pl.kernel,
      out_type=(
          jax.ShapeDtypeStruct((2 * _MP, _C), _f32),
          jax.ShapeDtypeStruct((2 * _MP,), _f32),
          jax.ShapeDtypeStruct((_EP,), _f32),
          jax.ShapeDtypeStruct((_EP,), _f32),
          jax.ShapeDtypeStruct((2 * _MP,), _f32),
      ),
      mesh=_sc_mesh(),
      compiler_params=pltpu.CompilerParams(needs_layout_passes=False),
      scratch_types=[
          pltpu.VMEM((16,), _f32),
          pltpu.VMEM((_CK,), _i32),
          pltpu.VMEM((_JS, 128), _i32),
          pltpu.VMEM((_CK,), _f32),
          pltpu.VMEM((_CK,), _f32),
          pltpu.VMEM((_CK,), _f32),
          pltpu.VMEM((_JS, 128), _f32),
          pltpu.VMEM((_CK,), _f32),
          pltpu.VMEM((128, _C), _f32),
          pltpu.VMEM((128, _C), _f32),
          pltpu.VMEM((_SL,), _f32),
          pltpu.VMEM_SHARED((_MP,), _f32),
          pltpu.VMEM_SHARED((_MP, _C), _f32),
          pltpu.SemaphoreType.DMA,
          pltpu.SemaphoreType.DMA,
          pltpu.SemaphoreType.DMA,
          pltpu.SemaphoreType.DMA,
      ],
  )
  def k(row_h, col_h, st_h, ax_h, ae_h, sh_h, xl_h, rc2_h,
        eo_h, denp_h, ex_h, w_h, deg_h,
        shb, ridx, cidx, stb, wb, axg, aeg, exb, rb0, rb1, tb,
        den, eos, lsem, ssem, gsem, wsem):
    c = lax.axis_index("c")
    s = lax.axis_index("s")
    iota = _iota16()
    rb = (rb0, rb1)
    w = c * _NS + s

    pltpu.sync_copy(sh_h, shb)
    sh = shb[...]

    if do_deg:
      cf = jnp.full((16,), (1 - c).astype(_f32), _f32)
      for g in range(_SL // 16):
        tb[pl.ds(g * 16, 16)] = _zero16()
      pltpu.sync_copy(tb, den.at[pl.ds(s * _SL, _SL)])
      plsc.subcore_barrier()

      def fire_d(base_e):
        pltpu.async_copy(st_h.at[pl.ds(base_e, _CK)], stb, lsem)
        for j in range(_JS):
          pltpu.async_copy(rc2_h.at[pl.ds(c * _EP + base_e + j * 128, 128)],
                           cidx.at[j], lsem)

      def drain_d(base_e):
        pltpu.make_async_copy(st_h.at[pl.ds(base_e, _CK)], stb, lsem).wait()
        for j in range(_JS):
          pltpu.make_async_copy(
              rc2_h.at[pl.ds(c * _EP + base_e + j * 128, 128)],
              cidx.at[j], lsem).wait()

      fire_d(s * _EA)

      def chunk_d(k_, _):
        base_e = s * _EA + k_ * _CK
        drain_d(base_e)
        for j in range(_JS):
          for g in range(8):
            off = j * 128 + g * 16
            msk = (base_e + off + iota) < _E
            sv = jnp.abs(stb[pl.ds(off, 16)])
            val = cf + (1.0 - cf) * sv
            wb[pl.ds(off, 16)] = jnp.where(msk, val, 0.0)
        sd = []
        for j in range(_JS):
          sd.append(pltpu.async_copy(wb.at[pl.ds(j * 128, 128)],
                                     den.at[cidx.at[j]], ssem, add=True))
        for d in sd:
          d.wait()
        nxt = s * _EA + jnp.minimum(k_ + 1, _CA - 1) * _CK
        fire_d(nxt)
        return 0

      lax.fori_loop(0, _CA, chunk_d, 0)
      drain_d(s * _EA + (_CA - 1) * _CK)
      plsc.subcore_barrier()

      pltpu.sync_copy(den.at[pl.ds(s * _SL, _SL)], tb)
      for g in range(_SL // 16):
        v = tb[pl.ds(g * 16, 16)]
        tb[pl.ds(g * 16, 16)] = jnp.where(v > 0, 1.0 / v, 0.0)
      pltpu.sync_copy(tb, deg_h.at[pl.ds(c * _MP + s * _SL, _SL)])

    for g in range(_SL // 16):
      tb[pl.ds(g * 16, 16)] = _zero16()
    pltpu.sync_copy(tb, den.at[pl.ds(s * _SL, _SL)])

    def zr(r, _):
      for v in range(8):
        rb0[r, pl.ds(v * 16, 16)] = _zero16()
      return 0

    lax.fori_loop(0, 128, zr, 0)
    for t in range(_SL // 128):
      pltpu.sync_copy(rb0, eos.at[pl.ds(s * _SL + t * 128, 128)])
    plsc.subcore_barrier()

    def fire_idx(base_e):
      pltpu.async_copy(row_h.at[pl.ds(base_e, _CK)], ridx, lsem)
      for j in range(_JS):
        pltpu.async_copy(col_h.at[pl.ds(base_e + j * 128, 128)],
                         cidx.at[j], lsem)

    def drain_idx(base_e):
      pltpu.make_async_copy(row_h.at[pl.ds(base_e, _CK)], ridx, lsem).wait()
      for j in range(_JS):
        pltpu.make_async_copy(col_h.at[pl.ds(base_e + j * 128, 128)],
                              cidx.at[j], lsem).wait()

    fire_idx(w * _EB)

    def chunk_a(k_, _):
      base_e = w * _EB + k_ * _CK
      drain_idx(base_e)
      gax = pltpu.async_copy(ax_h.at[ridx], axg, gsem)
      gae = []
      for j in range(_JS):
        gae.append(pltpu.async_copy(ae_h.at[cidx.at[j]], aeg.at[j], gsem))
      gax.wait()
      for d in gae:
        d.wait()
      for j in range(_JS):
        for g in range(8):
          off = j * 128 + g * 16
          a = axg[pl.ds(off, 16)] + aeg[j, pl.ds(g * 16, 16)]
          a = jnp.where(a >= 0, a, 0.2 * a)
          ex = jnp.exp(a - sh)
          msk = (base_e + off + iota) < _E
          wb[pl.ds(off, 16)] = jnp.where(msk, ex, 0.0)
      wd = pltpu.async_copy(wb, ex_h.at[pl.ds(base_e, _CK)], wsem)
      sd = []
      for j in range(_JS):
        sd.append(pltpu.async_copy(wb.at[pl.ds(j * 128, 128)],
                                   den.at[cidx.at[j]], ssem, add=True))
      for d in sd:
        d.wait()
      wd.wait()
      nxt = w * _EB + jnp.minimum(k_ + 1, _CB - 1) * _CK
      fire_idx(nxt)
      return 0

    lax.fori_loop(0, _CB, chunk_a, 0)
    drain_idx(w * _EB + (_CB - 1) * _CK)
    plsc.subcore_barrier()

    pltpu.sync_copy(den.at[pl.ds(s * _SL, _SL)],
                    denp_h.at[pl.ds(c * _MP + s * _SL, _SL)])

    def fire_loads_b(base_e):
      fire_idx(base_e)
      pltpu.async_copy(st_h.at[pl.ds(base_e, _CK)], stb, lsem)
      pltpu.async_copy(ex_h.at[pl.ds(base_e, _CK)], exb, lsem)

    def drain_loads_b(base_e):
      drain_idx(base_e)
      pltpu.make_async_copy(st_h.at[pl.ds(base_e, _CK)], stb, lsem).wait()
      pltpu.make_async_copy(ex_h.at[pl.ds(base_e, _CK)], exb, lsem).wait()

    fire_loads_b(w * _EB)

    def chunk_b(k_, _):
      base_e = w * _EB + k_ * _CK
      drain_loads_b(base_e)
      for j in range(_JS):
        for g in range(8):
          off = j * 128 + g * 16
          wb[pl.ds(off, 16)] = exb[pl.ds(off, 16)] * stb[pl.ds(off, 16)]
      wd = pltpu.async_copy(wb, w_h.at[pl.ds(base_e, _CK)], wsem)
      gd = [None] * _JS
      sd = [None] * _JS
      gd[0] = pltpu.async_copy(xl_h.at[ridx.at[pl.ds(0, 128)]], rb[0], gsem)
      for j in range(_JS):
        b = rb[j % 2]
        gd[j].wait()
        if j < _JS - 1:
          if j >= 1:
            sd[j - 1].wait()
          gd[j + 1] = pltpu.async_copy(
              xl_h.at[ridx.at[pl.ds((j + 1) * 128, 128)]], rb[(j + 1) % 2],
              gsem)

        def srow(t, _, _j=j, _b=b):
          r0 = t * 8
          for u in range(8):
            r = r0 + u
            wv = plsc.load_gather(wb, [jnp.full((16,), _j * 128 + r, _i32)])
            for v in range(8):
              _b[r, pl.ds(v * 16, 16)] = _b[r, pl.ds(v * 16, 16)] * wv
          return 0

        lax.fori_loop(0, 16, srow, 0)
        sd[j] = pltpu.async_copy(b, eos.at[cidx.at[j]], ssem, add=True)
      sd[_JS - 2].wait()
      sd[_JS - 1].wait()
      wd.wait()
      nxt = w * _EB + jnp.minimum(k_ + 1, _CB - 1) * _CK
      fire_loads_b(nxt)
      return 0

    lax.fori_loop(0, _CB, chunk_b, 0)
    drain_loads_b(w * _EB + (_CB - 1) * _CK)
    plsc.subcore_barrier()

    for t in range(_SL // 128):
      pltpu.sync_copy(eos.at[pl.ds(s * _SL + t * 128, 128)],
                      eo_h.at[pl.ds(c * _MP + s * _SL + t * 128, 128)])

  return k


def _sc_pass2():
  @functools.partial(
      pl.kernel,
      out_type=jax.ShapeDtypeStruct((2 * _MP, _C), _f32),
      mesh=_sc_mesh(),
      compiler_params=pltpu.CompilerParams(needs_layout_passes=False),
      scratch_types=[
          pltpu.VMEM((_CK,), _i32),
          pltpu.VMEM((_JS, 128), _i32),
          pltpu.VMEM((_CK,), _f32),
          pltpu.VMEM((128, _C), _f32),
          pltpu.VMEM((128, _C), _f32),
          pltpu.VMEM_SHARED((_MP, _C), _f32),
          pltpu.SemaphoreType.DMA,
          pltpu.SemaphoreType.DMA,
          pltpu.SemaphoreType.DMA,
      ],
  )
  def k(row_h, col_h, w_hbm, eo_h,
        o_h,
        cidx, ridx, wb, rb0, rb1,
        outs, lsem, ssem, gsem):
    c = lax.axis_index("c")
    s = lax.axis_index("s")
    rb = (rb0, rb1)
    w = c * _NS + s

    def zr(r, _):
      for v in range(8):
        rb0[r, pl.ds(v * 16, 16)] = _zero16()
      return 0

    lax.fori_loop(0, 128, zr, 0)
    for t in range(_SL // 128):
      pltpu.sync_copy(rb0, outs.at[pl.ds(s * _SL + t * 128, 128)])
    plsc.subcore_barrier()

    def fire_loads(base_e):
      pltpu.async_copy(col_h.at[pl.ds(base_e, _CK)], cidx, lsem)
      pltpu.async_copy(w_hbm.at[pl.ds(base_e, _CK)], wb, lsem)
      for j in range(_JS):
        pltpu.async_copy(row_h.at[pl.ds(base_e + j * 128, 128)],
                         ridx.at[j], lsem)

    def drain_loads(base_e):
      pltpu.make_async_copy(col_h.at[pl.ds(base_e, _CK)], cidx, lsem).wait()
      pltpu.make_async_copy(w_hbm.at[pl.ds(base_e, _CK)], wb, lsem).wait()
      for j in range(_JS):
        pltpu.make_async_copy(row_h.at[pl.ds(base_e + j * 128, 128)],
                              ridx.at[j], lsem).wait()

    fire_loads(w * _EB)

    def chunk(k_, _):
      base_e = w * _EB + k_ * _CK
      drain_loads(base_e)
      gd = [None] * _JS
      sd = [None] * _JS
      gd[0] = pltpu.async_copy(eo_h.at[cidx.at[pl.ds(0, 128)]], rb[0], gsem)
      for j in range(_JS):
        b = rb[j % 2]
        gd[j].wait()
        if j < _JS - 1:
          if j >= 1:
            sd[j - 1].wait()
          gd[j + 1] = pltpu.async_copy(
              eo_h.at[cidx.at[pl.ds((j + 1) * 128, 128)]], rb[(j + 1) % 2],
              gsem)

        def srow(t, _, _j=j, _b=b):
          r0 = t * 8
          for u in range(8):
            r = r0 + u
            wv = plsc.load_gather(wb, [jnp.full((16,), _j * 128 + r, _i32)])
            for v in range(8):
              _b[r, pl.ds(v * 16, 16)] = _b[r, pl.ds(v * 16, 16)] * wv
          return 0

        lax.fori_loop(0, 16, srow, 0)
        sd[j] = pltpu.async_copy(b, outs.at[ridx.at[j]], ssem, add=True)
      sd[_JS - 2].wait()
      sd[_JS - 1].wait()
      nxt = w * _EB + jnp.minimum(k_ + 1, _CB - 1) * _CK
      fire_loads(nxt)
      return 0

    lax.fori_loop(0, _CB, chunk, 0)
    drain_loads(w * _EB + (_CB - 1) * _CK)
    plsc.subcore_barrier()

    for t in range(_SL // 128):
      pltpu.sync_copy(outs.at[pl.ds(s * _SL + t * 128, 128)],
                      o_h.at[pl.ds(c * _MP + s * _SL + t * 128, 128)])

  return k


def kernel(metabolite_embeddings, hyperedge_index, stoichiometry,
           reaction_features, W1, att1, b1, g1, be1, W2, att2, b2, g2, be2):
  x = metabolite_embeddings.astype(_f32)
  hattr = reaction_features.astype(_f32)
  row = hyperedge_index[0].astype(_i32)
  col = hyperedge_index[1].astype(_i32)
  st = stoichiometry.astype(_f32)

  pad = _EP - _E
  pidx = jnp.arange(pad, dtype=_i32)
  row_p = jnp.concatenate([row, pidx % _N])
  col_p = jnp.concatenate([col, pidx % _M])
  st_p = jnp.concatenate([st, jnp.zeros((pad,), _f32)])
  rc2 = jnp.concatenate([col_p, row_p])

  binv2 = None
  dinv2 = None

  for lyr, (w, att, b, g, be) in enumerate(((W1, att1, b1, g1, be1),
                                            (W2, att2, b2, g2, be2))):
    att_a = att[:_C].reshape(1, _C).astype(_f32)
    att_b = att[_C:].reshape(1, _C).astype(_f32)
    xl, axv, aev, shiftv = _tc_pre(x, hattr, w.astype(_f32), att_a, att_b)
    axv = axv.reshape(_N)
    aev = aev.reshape(_M)
    s1, den_p, _exu, wv_e, deg = _sc_pass1(lyr == 0)(
        row_p, col_p, st_p, axv, aev, shiftv, xl, rc2)
    if lyr == 0:
      binv2 = deg[:_MP].reshape(_MP, 1)
      dinv2 = deg[_MP:_MP + _N].reshape(_N, 1)
    eoq = _tc_combine(s1, den_p.reshape(2 * _MP, 1), binv2)
    oo2 = _sc_pass2()(row_p, col_p, wv_e, eoq)
    x = _tc_post(oo2, dinv2, b.reshape(1, _C).astype(_f32),
                 g.reshape(1, _C).astype(_f32),
                 be.reshape(1, _C).astype(_f32), x)
  return x


# speedup vs baseline: 1.0208x; 1.0208x over previous
import functools

import jax
import jax.numpy as jnp
from jax import lax
from jax.experimental import pallas as pl
from jax.experimental.pallas import tpu as pltpu
from jax.experimental.pallas import tpu_sc as plsc

_N = 10000
_M = 10000
_E = 320000
_C = 128

_NC = 2
_NS = 16
_NW = _NC * _NS

_MP = 10240
_SL = _MP // _NS
_EP = 327680
_EB = _EP // _NW
_CK = 1024
_JS = _CK // 128
_CB = _EB // _CK
_EA = _EP // _NS
_CA = _EA // _CK

_f32 = jnp.float32
_i32 = jnp.int32


def _iota16():
  return lax.broadcasted_iota(_i32, (16,), 0)


def _zero16():
  return jnp.zeros((16,), _f32)


def _tc_pre(x, hattr, w, att_a, att_b):
  blk = 400
  nb = _N // blk

  def body(x_ref, h_ref, w_ref, aa_ref, ab_ref, xl_ref, ax_ref, ae_ref,
           sh_ref, smem):
    i = pl.program_id(0)

    @pl.when(i == 0)
    def _():
      smem[0] = -1e30
      smem[1] = -1e30

    wv = w_ref[...]
    xb = jnp.dot(x_ref[...], wv, preferred_element_type=_f32)
    axb = jnp.sum(xb * aa_ref[...], axis=1)
    eb = jnp.dot(h_ref[...], wv, preferred_element_type=_f32)
    aeb = jnp.sum(eb * ab_ref[...], axis=1)
    xl_ref[...] = xb
    ax_ref[...] = axb[:, None]
    ae_ref[...] = aeb[:, None]
    smem[0] = jnp.maximum(smem[0], jnp.max(axb))
    smem[1] = jnp.maximum(smem[1], jnp.max(aeb))

    @pl.when(i == nb - 1)
    def _():
      sh_ref[...] = jnp.full((16,), jnp.maximum(smem[0] + smem[1], 0.0), _f32)

  return pl.pallas_call(
      body,
      grid=(nb,),
      in_specs=[
          pl.BlockSpec((blk, _C), lambda i: (i, 0)),
          pl.BlockSpec((blk, _C), lambda i: (i, 0)),
          pl.BlockSpec((_C, _C), lambda i: (0, 0)),
          pl.BlockSpec((1, _C), lambda i: (0, 0)),
          pl.BlockSpec((1, _C), lambda i: (0, 0)),
      ],
      out_specs=[
          pl.BlockSpec((blk, _C), lambda i: (i, 0)),
          pl.BlockSpec((blk, 1), lambda i: (i, 0)),
          pl.BlockSpec((blk, 1), lambda i: (i, 0)),
          pl.BlockSpec((16,), lambda i: (0,)),
      ],
      out_shape=[
          jax.ShapeDtypeStruct((_N, _C), _f32),
          jax.ShapeDtypeStruct((_N, 1), _f32),
          jax.ShapeDtypeStruct((_M, 1), _f32),
          jax.ShapeDtypeStruct((16,), _f32),
      ],
      scratch_shapes=[pltpu.SMEM((2,), _f32)],
  )(x, hattr, w, att_a, att_b)


def _tc_combine(parts, den2, binv2):
  blk = 80
  nb = _MP // blk

  def body(a_ref, b_ref, da_ref, db_ref, bi_ref, o_ref):
    q = 1.0 / (da_ref[...] + db_ref[...] + 1e-16)
    o_ref[...] = (a_ref[...] + b_ref[...]) * (bi_ref[...] * q * q)

  return pl.pallas_call(
      body,
      grid=(nb,),
      in_specs=[
          pl.BlockSpec((blk, _C), lambda i: (i, 0)),
          pl.BlockSpec((blk, _C), lambda i: (i + nb, 0)),
          pl.BlockSpec((blk, 1), lambda i: (i, 0)),
          pl.BlockSpec((blk, 1), lambda i: (i + nb, 0)),
          pl.BlockSpec((blk, 1), lambda i: (i, 0)),
      ],
      out_specs=pl.BlockSpec((blk, _C), lambda i: (i, 0)),
      out_shape=jax.ShapeDtypeStruct((_MP, _C), _f32),
  )(parts, parts, den2, den2, binv2)


def _tc_post(parts, dinv2, bvec, gvec, bevec, x):
  blk = 80
  nb = _MP // blk

  def body(a_ref, b_ref, di_ref, bv_ref, g_ref, be_ref, x_ref, o_ref):
    o = (a_ref[...] + b_ref[...]) * di_ref[...] + bv_ref[...]
    m = jnp.mean(o, axis=1, keepdims=True)
    d = o - m
    v = jnp.mean(d * d, axis=1, keepdims=True)
    ln = d / jnp.sqrt(v + 1e-5) * g_ref[...] + be_ref[...]
    o_ref[...] = jnp.tanh(ln + x_ref[...])

  return pl.pallas_call(
      body,
      grid=(_N // blk,),
      in_specs=[
          pl.BlockSpec((blk, _C), lambda i: (i, 0)),
          pl.BlockSpec((blk, _C), lambda i: (i + nb, 0)),
          pl.BlockSpec((blk, 1), lambda i: (i, 0)),
          pl.BlockSpec((1, _C), lambda i: (0, 0)),
          pl.BlockSpec((1, _C), lambda i: (0, 0)),
          pl.BlockSpec((1, _C), lambda i: (0, 0)),
          pl.BlockSpec((blk, _C), lambda i: (i, 0)),
      ],
      out_specs=pl.BlockSpec((blk, _C), lambda i: (i, 0)),
      out_shape=jax.ShapeDtypeStruct((_N, _C), _f32),
  )(parts, parts, dinv2, bvec, gvec, bevec, x)


def _sc_mesh():
  return plsc.VectorSubcoreMesh(core_axis_name="c", subcore_axis_name="s")


def _sc_degrees():
  @functools.partial(
      pl.kernel,
      out_type=jax.ShapeDtypeStruct((2 * _MP,), _f32),
      mesh=_sc_mesh(),
      compiler_params=pltpu.CompilerParams(needs_layout_passes=False),
      scratch_types=[
          pltpu.VMEM((_JS, 128), _i32),
          pltpu.VMEM((_CK,), _f32),
          pltpu.VMEM((_JS, 128), _f32),
          pltpu.VMEM((_SL,), _f32),
          pltpu.VMEM_SHARED((_MP,), _f32),
          pltpu.SemaphoreType.DMA,
          pltpu.SemaphoreType.DMA,
      ],
  )
  def k(rc2_h, st2_h, out_h, idxb, stb, updb, tbuf, acc, lsem, ssem):
    c = lax.axis_index("c")
    s = lax.axis_index("s")
    iota = _iota16()
    cf = jnp.full((16,), (1 - c).astype(_f32), _f32)

    for g in range(_SL // 16):
      tbuf[pl.ds(g * 16, 16)] = _zero16()
    pltpu.sync_copy(tbuf, acc.at[pl.ds(s * _SL, _SL)])
    plsc.subcore_barrier()

    def fire_loads(base_e):
      pltpu.async_copy(st2_h.at[pl.ds(base_e, _CK)], stb, lsem)
      for j in range(_JS):
        pltpu.async_copy(rc2_h.at[pl.ds(c * _EP + base_e + j * 128, 128)],
                         idxb.at[j], lsem)

    def drain_loads(base_e):
      pltpu.make_async_copy(st2_h.at[pl.ds(base_e, _CK)], stb, lsem).wait()
      for j in range(_JS):
        pltpu.make_async_copy(rc2_h.at[pl.ds(c * _EP + base_e + j * 128, 128)],
                              idxb.at[j], lsem).wait()

    fire_loads(s * _EA)

    def chunk(k_, _):
      base_e = s * _EA + k_ * _CK
      drain_loads(base_e)
      for j in range(_JS):
        for g in range(8):
          off = j * 128 + g * 16
          msk = (base_e + off + iota) < _E
          sv = jnp.abs(stb[pl.ds(off, 16)])
          val = cf + (1.0 - cf) * sv
          updb[j, pl.ds(g * 16, 16)] = jnp.where(msk, val, 0.0)
      sd = []
      for j in range(_JS):
        sd.append(pltpu.async_copy(updb.at[j], acc.at[idxb.at[j]], ssem,
                                   add=True))
      for d in sd:
        d.wait()
      nxt = s * _EA + jnp.minimum(k_ + 1, _CA - 1) * _CK
      fire_loads(nxt)
      return 0

    lax.fori_loop(0, _CA, chunk, 0)
    drain_loads(s * _EA + (_CA - 1) * _CK)
    plsc.subcore_barrier()

    pltpu.sync_copy(acc.at[pl.ds(s * _SL, _SL)], tbuf)
    for g in range(_SL // 16):
      v = tbuf[pl.ds(g * 16, 16)]
      tbuf[pl.ds(g * 16, 16)] = jnp.where(v > 0, 1.0 / v, 0.0)
    pltpu.sync_copy(tbuf, out_h.at[pl.ds(c * _MP + s * _SL, _SL)])

  return k


def _sc_pass1():
  @functools.partial(
      pl.kernel,
      out_type=(
          jax.ShapeDtypeStruct((2 * _MP, _C), _f32),
          jax.ShapeDtypeStruct((2 * _MP,), _f32),
          jax.ShapeDtypeStruct((_EP,), _f32),
          jax.ShapeDtypeStruct((_EP,), _f32),
      ),
      mesh=_sc_mesh(),
      compiler_params=pltpu.CompilerParams(needs_layout_passes=False),
      scratch_types=[
          pltpu.VMEM((16,), _f32),
          pltpu.VMEM((_CK,), _i32),
          pltpu.VMEM((_JS, 128), _i32),
          pltpu.VMEM((_CK,), _f32),
          pltpu.VMEM((_CK,), _f32),
          pltpu.VMEM((_CK,), _f32),
          pltpu.VMEM((_JS, 128), _f32),
          pltpu.VMEM((_CK,), _f32),
          pltpu.VMEM((128, _C), _f32),
          pltpu.VMEM((128, _C), _f32),
          pltpu.VMEM((_SL,), _f32),
          pltpu.VMEM_SHARED((_MP,), _f32),
          pltpu.VMEM_SHARED((_MP, _C), _f32),
          pltpu.SemaphoreType.DMA,
          pltpu.SemaphoreType.DMA,
          pltpu.SemaphoreType.DMA,
          pltpu.SemaphoreType.DMA,
      ],
  )
  def k(row_h, col_h, st_h, ax_h, ae_h, sh_h, xl_h,
        eo_h, denp_h, ex_h, w_h,
        shb, ridx, cidx, stb, wb, axg, aeg, exb, rb0, rb1, tb,
        den, eos, lsem, ssem, gsem, wsem):
    c = lax.axis_index("c")
    s = lax.axis_index("s")
    iota = _iota16()
    rb = (rb0, rb1)
    w = c * _NS + s

    pltpu.sync_copy(sh_h, shb)
    sh = shb[...]

    for g in range(_SL // 16):
      tb[pl.ds(g * 16, 16)] = _zero16()
    pltpu.sync_copy(tb, den.at[pl.ds(s * _SL, _SL)])

    def zr(r, _):
      for v in range(8):
        rb0[r, pl.ds(v * 16, 16)] = _zero16()
      return 0

    lax.fori_loop(0, 128, zr, 0)
    for t in range(_SL // 128):
      pltpu.sync_copy(rb0, eos.at[pl.ds(s * _SL + t * 128, 128)])
    plsc.subcore_barrier()

    def fire_idx(base_e):
      pltpu.async_copy(row_h.at[pl.ds(base_e, _CK)], ridx, lsem)
      for j in range(_JS):
        pltpu.async_copy(col_h.at[pl.ds(base_e + j * 128, 128)],
                         cidx.at[j], lsem)

    def drain_idx(base_e):
      pltpu.make_async_copy(row_h.at[pl.ds(base_e, _CK)], ridx, lsem).wait()
      for j in range(_JS):
        pltpu.make_async_copy(col_h.at[pl.ds(base_e + j * 128, 128)],
                              cidx.at[j], lsem).wait()

    fire_idx(w * _EB)

    def chunk_a(k_, _):
      base_e = w * _EB + k_ * _CK
      drain_idx(base_e)
      gax = pltpu.async_copy(ax_h.at[ridx], axg, gsem)
      gae = []
      for j in range(_JS):
        gae.append(pltpu.async_copy(ae_h.at[cidx.at[j]], aeg.at[j], gsem))
      gax.wait()
      for d in gae:
        d.wait()
      for j in range(_JS):
        for g in range(8):
          off = j * 128 + g * 16
          a = axg[pl.ds(off, 16)] + aeg[j, pl.ds(g * 16, 16)]
          a = jnp.where(a >= 0, a, 0.2 * a)
          ex = jnp.exp(a - sh)
          msk = (base_e + off + iota) < _E
          wb[pl.ds(off, 16)] = jnp.where(msk, ex, 0.0)
      wd = pltpu.async_copy(wb, ex_h.at[pl.ds(base_e, _CK)], wsem)
      sd = []
      for j in range(_JS):
        sd.append(pltpu.async_copy(wb.at[pl.ds(j * 128, 128)],
                                   den.at[cidx.at[j]], ssem, add=True))
      for d in sd:
        d.wait()
      wd.wait()
      nxt = w * _EB + jnp.minimum(k_ + 1, _CB - 1) * _CK
      fire_idx(nxt)
      return 0

    lax.fori_loop(0, _CB, chunk_a, 0)
    drain_idx(w * _EB + (_CB - 1) * _CK)
    plsc.subcore_barrier()

    pltpu.sync_copy(den.at[pl.ds(s * _SL, _SL)],
                    denp_h.at[pl.ds(c * _MP + s * _SL, _SL)])

    def fire_loads_b(base_e):
      fire_idx(base_e)
      pltpu.async_copy(st_h.at[pl.ds(base_e, _CK)], stb, lsem)
      pltpu.async_copy(ex_h.at[pl.ds(base_e, _CK)], exb, lsem)

    def drain_loads_b(base_e):
      drain_idx(base_e)
      pltpu.make_async_copy(st_h.at[pl.ds(base_e, _CK)], stb, lsem).wait()
      pltpu.make_async_copy(ex_h.at[pl.ds(base_e, _CK)], exb, lsem).wait()

    fire_loads_b(w * _EB)

    def chunk_b(k_, _):
      base_e = w * _EB + k_ * _CK
      drain_loads_b(base_e)
      for j in range(_JS):
        for g in range(8):
          off = j * 128 + g * 16
          wb[pl.ds(off, 16)] = exb[pl.ds(off, 16)] * stb[pl.ds(off, 16)]
      wd = pltpu.async_copy(wb, w_h.at[pl.ds(base_e, _CK)], wsem)
      gd = [None] * _JS
      sd = [None] * _JS
      gd[0] = pltpu.async_copy(xl_h.at[ridx.at[pl.ds(0, 128)]], rb[0], gsem)
      for j in range(_JS):
        b = rb[j % 2]
        gd[j].wait()
        if j < _JS - 1:
          if j >= 1:
            sd[j - 1].wait()
          gd[j + 1] = pltpu.async_copy(
              xl_h.at[ridx.at[pl.ds((j + 1) * 128, 128)]], rb[(j + 1) % 2],
              gsem)

        def srow(t, _, _j=j, _b=b):
          r0 = t * 8
          for u in range(8):
            r = r0 + u
            wv = plsc.load_gather(wb, [jnp.full((16,), _j * 128 + r, _i32)])
            for v in range(8):
              _b[r, pl.ds(v * 16, 16)] = _b[r, pl.ds(v * 16, 16)] * wv
          return 0

        lax.fori_loop(0, 16, srow, 0)
        sd[j] = pltpu.async_copy(b, eos.at[cidx.at[j]], ssem, add=True)
      nxt = w * _EB + jnp.minimum(k_ + 1, _CB - 1) * _CK
      pltpu.async_copy(row_h.at[pl.ds(nxt, _CK)], ridx, lsem)
      pltpu.async_copy(st_h.at[pl.ds(nxt, _CK)], stb, lsem)
      pltpu.async_copy(ex_h.at[pl.ds(nxt, _CK)], exb, lsem)
      sd[_JS - 2].wait()
      sd[_JS - 1].wait()
      wd.wait()
      for j in range(_JS):
        pltpu.async_copy(col_h.at[pl.ds(nxt + j * 128, 128)],
                         cidx.at[j], lsem)
      return 0

    lax.fori_loop(0, _CB, chunk_b, 0)
    drain_loads_b(w * _EB + (_CB - 1) * _CK)
    plsc.subcore_barrier()

    for t in range(_SL // 128):
      pltpu.sync_copy(eos.at[pl.ds(s * _SL + t * 128, 128)],
                      eo_h.at[pl.ds(c * _MP + s * _SL + t * 128, 128)])

  return k


def _sc_pass2():
  @functools.partial(
      pl.kernel,
      out_type=jax.ShapeDtypeStruct((2 * _MP, _C), _f32),
      mesh=_sc_mesh(),
      compiler_params=pltpu.CompilerParams(needs_layout_passes=False),
      scratch_types=[
          pltpu.VMEM((_CK,), _i32),
          pltpu.VMEM((_JS, 128), _i32),
          pltpu.VMEM((_CK,), _f32),
          pltpu.VMEM((128, _C), _f32),
          pltpu.VMEM((128, _C), _f32),
          pltpu.VMEM_SHARED((_MP, _C), _f32),
          pltpu.SemaphoreType.DMA,
          pltpu.SemaphoreType.DMA,
          pltpu.SemaphoreType.DMA,
      ],
  )
  def k(row_h, col_h, w_hbm, eo_h,
        o_h,
        cidx, ridx, wb, rb0, rb1,
        outs, lsem, ssem, gsem):
    c = lax.axis_index("c")
    s = lax.axis_index("s")
    rb = (rb0, rb1)
    w = c * _NS + s

    def zr(r, _):
      for v in range(8):
        rb0[r, pl.ds(v * 16, 16)] = _zero16()
      return 0

    lax.fori_loop(0, 128, zr, 0)
    for t in range(_SL // 128):
      pltpu.sync_copy(rb0, outs.at[pl.ds(s * _SL + t * 128, 128)])
    plsc.subcore_barrier()

    def fire_loads(base_e):
      pltpu.async_copy(col_h.at[pl.ds(base_e, _CK)], cidx, lsem)
      pltpu.async_copy(w_hbm.at[pl.ds(base_e, _CK)], wb, lsem)
      for j in range(_JS):
        pltpu.async_copy(row_h.at[pl.ds(base_e + j * 128, 128)],
                         ridx.at[j], lsem)

    def drain_loads(base_e):
      pltpu.make_async_copy(col_h.at[pl.ds(base_e, _CK)], cidx, lsem).wait()
      pltpu.make_async_copy(w_hbm.at[pl.ds(base_e, _CK)], wb, lsem).wait()
      for j in range(_JS):
        pltpu.make_async_copy(row_h.at[pl.ds(base_e + j * 128, 128)],
                              ridx.at[j], lsem).wait()

    fire_loads(w * _EB)

    def chunk(k_, _):
      base_e = w * _EB + k_ * _CK
      drain_loads(base_e)
      gd = [None] * _JS
      sd = [None] * _JS
      gd[0] = pltpu.async_copy(eo_h.at[cidx.at[pl.ds(0, 128)]], rb[0], gsem)
      for j in range(_JS):
        b = rb[j % 2]
        gd[j].wait()
        if j < _JS - 1:
          if j >= 1:
            sd[j - 1].wait()
          gd[j + 1] = pltpu.async_copy(
              eo_h.at[cidx.at[pl.ds((j + 1) * 128, 128)]], rb[(j + 1) % 2],
              gsem)

        def srow(t, _, _j=j, _b=b):
          r0 = t * 8
          for u in range(8):
            r = r0 + u
            wv = plsc.load_gather(wb, [jnp.full((16,), _j * 128 + r, _i32)])
            for v in range(8):
              _b[r, pl.ds(v * 16, 16)] = _b[r, pl.ds(v * 16, 16)] * wv
          return 0

        lax.fori_loop(0, 16, srow, 0)
        sd[j] = pltpu.async_copy(b, outs.at[ridx.at[j]], ssem, add=True)
      nxt = w * _EB + jnp.minimum(k_ + 1, _CB - 1) * _CK
      pltpu.async_copy(col_h.at[pl.ds(nxt, _CK)], cidx, lsem)
      pltpu.async_copy(w_hbm.at[pl.ds(nxt, _CK)], wb, lsem)
      sd[_JS - 2].wait()
      sd[_JS - 1].wait()
      for j in range(_JS):
        pltpu.async_copy(row_h.at[pl.ds(nxt + j * 128, 128)],
                         ridx.at[j], lsem)
      return 0

    lax.fori_loop(0, _CB, chunk, 0)
    drain_loads(w * _EB + (_CB - 1) * _CK)
    plsc.subcore_barrier()

    for t in range(_SL // 128):
      pltpu.sync_copy(outs.at[pl.ds(s * _SL + t * 128, 128)],
                      o_h.at[pl.ds(c * _MP + s * _SL + t * 128, 128)])

  return k


def kernel(metabolite_embeddings, hyperedge_index, stoichiometry,
           reaction_features, W1, att1, b1, g1, be1, W2, att2, b2, g2, be2):
  x = metabolite_embeddings.astype(_f32)
  hattr = reaction_features.astype(_f32)
  row = hyperedge_index[0].astype(_i32)
  col = hyperedge_index[1].astype(_i32)
  st = stoichiometry.astype(_f32)

  pad = _EP - _E
  pidx = jnp.arange(pad, dtype=_i32)
  row_p = jnp.concatenate([row, pidx % _N])
  col_p = jnp.concatenate([col, pidx % _M])
  st_p = jnp.concatenate([st, jnp.zeros((pad,), _f32)])
  rc2 = jnp.concatenate([col_p, row_p])

  deg = _sc_degrees()(rc2, st_p)
  binv2 = deg[:_MP].reshape(_MP, 1)
  dinv2 = deg[_MP:_MP + _N].reshape(_N, 1)

  for (w, att, b, g, be) in ((W1, att1, b1, g1, be1),
                             (W2, att2, b2, g2, be2)):
    att_a = att[:_C].reshape(1, _C).astype(_f32)
    att_b = att[_C:].reshape(1, _C).astype(_f32)
    xl, axv, aev, shiftv = _tc_pre(x, hattr, w.astype(_f32), att_a, att_b)
    axv = axv.reshape(_N)
    aev = aev.reshape(_M)
    s1, den_p, _exu, wv_e = _sc_pass1()(row_p, col_p, st_p, axv, aev,
                                        shiftv, xl)
    eoq = _tc_combine(s1, den_p.reshape(2 * _MP, 1), binv2)
    oo2 = _sc_pass2()(row_p, col_p, wv_e, eoq)
    x = _tc_post(oo2, dinv2, b.reshape(1, _C).astype(_f32),
                 g.reshape(1, _C).astype(_f32),
                 be.reshape(1, _C).astype(_f32), x)
  return x
